# Initial kernel scaffold; baseline (speedup 1.0000x reference)
import jax
import jax.numpy as jnp
from jax.experimental import pallas as pl


def kernel(vocab_word_idx, vocab_embedding_table):
    raise NotImplementedError("write your pallas kernel here")


# speedup vs baseline: 1.8796x; 1.8796x over previous
import functools

import jax
import jax.numpy as jnp
from jax import lax
from jax.experimental import pallas as pl
from jax.experimental.pallas import tpu as pltpu
from jax.experimental.pallas import tpu_sc as plsc

_VOCAB = 1000000
_EMBED = 64
_BATCH = 16384
_SEQ = 50

_NC = 2
_NS = 16
_NW = _NC * _NS
_TOT = _BATCH * _SEQ
_BPW = _TOT // _NW
_CHUNK = 128
_NCH = _BPW // _CHUNK
_NBUF = 4
_NGRP = _NCH // _NBUF

_mesh = plsc.VectorSubcoreMesh(core_axis_name="c", subcore_axis_name="s")


@functools.partial(
    pl.kernel,
    mesh=_mesh,
    out_type=jax.ShapeDtypeStruct((_TOT, _EMBED), jnp.float32),
    scratch_types=(
        [pltpu.VMEM((_NCH, _CHUNK), jnp.int32)]
        + [pltpu.VMEM((_CHUNK, _EMBED), jnp.float32)] * _NBUF
        + [pltpu.SemaphoreType.DMA] * _NBUF
    ),
    compiler_params=pltpu.CompilerParams(use_tc_tiling_on_sc=False),
)
def _sc_gather(idx_hbm, table_hbm, out_hbm,
               idx_v, r0, r1, r2, r3, s0, s1, s2, s3):
    rows = (r0, r1, r2, r3)
    sems = (s0, s1, s2, s3)
    wid = lax.axis_index("s") * _NC + lax.axis_index("c")
    base = wid * _BPW

    pltpu.sync_copy(idx_hbm.at[wid], idx_v)

    def _start(c, b):
        pltpu.make_async_copy(table_hbm.at[idx_v.at[c]], rows[b], sems[b]).start()

    def _finish(c, b):
        pltpu.make_async_copy(table_hbm.at[idx_v.at[c]], rows[b], sems[b]).wait()
        pltpu.sync_copy(rows[b], out_hbm.at[pl.ds(base + c * _CHUNK, _CHUNK)])

    for b in range(_NBUF):
        _start(b, b)

    def _group(g, carry):
        for b in range(_NBUF):
            c = g * _NBUF + b
            _finish(c, b)
            _start(c + _NBUF, b)
        return carry

    lax.fori_loop(0, _NGRP - 1, _group, 0)

    for b in range(_NBUF):
        _finish((_NGRP - 1) * _NBUF + b, b)


def kernel(vocab_word_idx, vocab_embedding_table):
    idx = vocab_word_idx.astype(jnp.int32).reshape(_NW, _NCH, _CHUNK)
    out = _sc_gather(idx, vocab_embedding_table)
    return out.reshape(_BATCH, _SEQ, _EMBED)
